# baseline (speedup 1.0000x reference)
import functools

import jax
import jax.numpy as jnp
from jax import lax
from jax.experimental import pallas as pl
from jax.experimental.pallas import tpu as pltpu
from jax.experimental.pallas import tpu_sc as plsc

_NUM_CORES = 2
_NUM_SUBCORES = 16
_NW = _NUM_CORES * _NUM_SUBCORES
_CHUNK = 248
_NCHUNKS = 126
_STRIPE = _CHUNK * _NCHUNKS
_NBUF = 4
_HALF = 2
_NGROUPS = 31
_TAIL_BASE = _NW * _STRIPE
_TAIL = 64


def kernel(embeds):
    rows, cols = embeds.shape
    mesh = plsc.VectorSubcoreMesh(core_axis_name="c", subcore_axis_name="s")

    @functools.partial(
        pl.kernel,
        mesh=mesh,
        out_type=jax.ShapeDtypeStruct((rows, cols), embeds.dtype),
        compiler_params=pltpu.CompilerParams(use_tc_tiling_on_sc=False),
        scratch_types=[
            pltpu.VMEM((_NBUF, _CHUNK, 64), jnp.float32),
            pltpu.SemaphoreType.DMA((_NBUF,)),
            pltpu.SemaphoreType.DMA((_NBUF,)),
        ],
    )
    def copy_kernel(in_hbm, out_hbm, buf, sem_in, sem_out):
        wid = lax.axis_index("s") * _NUM_CORES + lax.axis_index("c")
        wbase = pl.multiple_of(wid * _STRIPE, 8)

        def in_copy(k, b):
            base = pl.multiple_of(wbase + k * _CHUNK, 8)
            return pltpu.make_async_copy(
                in_hbm.at[pl.ds(base, _CHUNK)], buf.at[b], sem_in.at[b]
            )

        def out_copy(k, b):
            base = pl.multiple_of(wbase + k * _CHUNK, 8)
            return pltpu.make_async_copy(
                buf.at[b], out_hbm.at[pl.ds(base, _CHUNK)], sem_out.at[b]
            )

        in_copy(0, 0).start()
        in_copy(1, 1).start()

        def group(g, carry):
            for t in range(_NBUF):
                k = g * _NBUF + t
                in_copy(k, t).wait()
                out_copy(k, t).start()
                bp = (t + _HALF) % _NBUF
                kp = k + _HALF

                @pl.when(k >= _HALF)
                def _drain_prev_out():
                    out_copy(kp - _NBUF, bp).wait()

                in_copy(kp, bp).start()
            return carry

        lax.fori_loop(0, _NGROUPS, group, 0)

        in_copy(124, 0).wait()
        out_copy(124, 0).start()
        out_copy(122, 2).wait()
        in_copy(125, 1).wait()
        out_copy(125, 1).start()
        out_copy(123, 3).wait()
        out_copy(124, 0).wait()
        out_copy(125, 1).wait()

        @pl.when(wid == 0)
        def _copy_tail():
            pltpu.sync_copy(
                in_hbm.at[pl.ds(_TAIL_BASE, _TAIL)],
                buf.at[0, pl.ds(0, _TAIL)],
            )
            pltpu.sync_copy(
                buf.at[0, pl.ds(0, _TAIL)],
                out_hbm.at[pl.ds(_TAIL_BASE, _TAIL)],
            )

    return copy_kernel(embeds)


# speedup vs baseline: 1.3224x; 1.3224x over previous
import functools

import jax
import jax.numpy as jnp
from jax import lax
from jax.experimental import pallas as pl
from jax.experimental.pallas import tpu as pltpu
from jax.experimental.pallas import tpu_sc as plsc

_NUM_CORES = 2
_NUM_SUBCORES = 16
_NW = _NUM_CORES * _NUM_SUBCORES
_CHUNK = 248
_NCHUNKS = 126
_STRIPE = _CHUNK * _NCHUNKS
_NBUF = 4
_HALF = 2
_NGROUPS = 31
_TAIL_BASE = _NW * _STRIPE
_TAIL = 64


def kernel(embeds):
    rows, cols = embeds.shape
    mesh = plsc.VectorSubcoreMesh(core_axis_name="c", subcore_axis_name="s")

    @functools.partial(
        pl.kernel,
        mesh=mesh,
        out_type=jax.ShapeDtypeStruct((rows, cols), embeds.dtype),
        scratch_types=[
            pltpu.VMEM((_NBUF, _CHUNK, 64), jnp.float32),
            pltpu.SemaphoreType.DMA((_NBUF,)),
            pltpu.SemaphoreType.DMA((_NBUF,)),
        ],
    )
    def copy_kernel(in_hbm, out_hbm, buf, sem_in, sem_out):
        wid = lax.axis_index("s") * _NUM_CORES + lax.axis_index("c")
        wbase = pl.multiple_of(wid * _STRIPE, 8)

        def in_copy(k, b):
            base = pl.multiple_of(wbase + k * _CHUNK, 8)
            return pltpu.make_async_copy(
                in_hbm.at[pl.ds(base, _CHUNK)], buf.at[b], sem_in.at[b]
            )

        def out_copy(k, b):
            base = pl.multiple_of(wbase + k * _CHUNK, 8)
            return pltpu.make_async_copy(
                buf.at[b], out_hbm.at[pl.ds(base, _CHUNK)], sem_out.at[b]
            )

        in_copy(0, 0).start()
        in_copy(1, 1).start()

        def group(g, carry):
            for t in range(_NBUF):
                k = g * _NBUF + t
                in_copy(k, t).wait()
                out_copy(k, t).start()
                bp = (t + _HALF) % _NBUF
                kp = k + _HALF

                @pl.when(k >= _HALF)
                def _drain_prev_out():
                    out_copy(kp - _NBUF, bp).wait()

                in_copy(kp, bp).start()
            return carry

        lax.fori_loop(0, _NGROUPS, group, 0)

        in_copy(124, 0).wait()
        out_copy(124, 0).start()
        out_copy(122, 2).wait()
        in_copy(125, 1).wait()
        out_copy(125, 1).start()
        out_copy(123, 3).wait()
        out_copy(124, 0).wait()
        out_copy(125, 1).wait()

        @pl.when(wid == 0)
        def _copy_tail():
            pltpu.sync_copy(
                in_hbm.at[pl.ds(_TAIL_BASE, _TAIL)],
                buf.at[0, pl.ds(0, _TAIL)],
            )
            pltpu.sync_copy(
                buf.at[0, pl.ds(0, _TAIL)],
                out_hbm.at[pl.ds(_TAIL_BASE, _TAIL)],
            )

    return copy_kernel(embeds)
